# initial kernel scaffold (unmeasured)
import jax
import jax.numpy as jnp
from jax import lax
from jax.experimental import pallas as pl
from jax.experimental.pallas import tpu as pltpu

T = 1024
D = 2048
VH = 16384
NCHUNK = 8
CW = VH // NCHUNK
RB = 128


def _gemm(x, W):

    def body(x_ref, w_ref, o_ref):
        xb = x_ref[...].astype(jnp.bfloat16)
        wb = w_ref[...].astype(jnp.bfloat16)
        acc = jnp.dot(xb, wb, preferred_element_type=jnp.float32)
        o_ref[...] = acc.astype(jnp.bfloat16)

    return pl.pallas_call(
        body,
        grid=(NCHUNK,),
        in_specs=[
            pl.BlockSpec((T, D), lambda j: (0, 0)),
            pl.BlockSpec((D, CW), lambda j: (0, j)),
        ],
        out_specs=pl.BlockSpec((T, CW), lambda j: (0, j)),
        out_shape=jax.ShapeDtypeStruct((T, VH), jnp.bfloat16),
    )(x, W)


def _exchange(logits):

    def body(l_ref, o_ref, send_sem, recv_sem):
        my_x = lax.axis_index("x")
        my_y = lax.axis_index("y")
        my_z = lax.axis_index("z")
        peer = (my_x, my_y, 1 - my_z)

        barrier = pltpu.get_barrier_semaphore()
        pl.semaphore_signal(
            barrier, inc=1, device_id=peer, device_id_type=pl.DeviceIdType.MESH
        )
        pl.semaphore_wait(barrier, 1)

        rdma = pltpu.make_async_remote_copy(
            src_ref=l_ref,
            dst_ref=o_ref,
            send_sem=send_sem,
            recv_sem=recv_sem,
            device_id=peer,
            device_id_type=pl.DeviceIdType.MESH,
        )
        rdma.start()
        rdma.wait()

    return pl.pallas_call(
        body,
        out_shape=jax.ShapeDtypeStruct((T, VH), logits.dtype),
        in_specs=[pl.BlockSpec(memory_space=pltpu.ANY)],
        out_specs=pl.BlockSpec(memory_space=pltpu.ANY),
        scratch_shapes=[pltpu.SemaphoreType.DMA, pltpu.SemaphoreType.DMA],
        compiler_params=pltpu.CompilerParams(collective_id=0),
    )(logits)


def _softmax(mine, other):

    def body(a_ref, b_ref, o_ref):
        my_z = lax.axis_index("z")
        a = a_ref[...].astype(jnp.float32)
        b = b_ref[...].astype(jnp.float32)
        m = jnp.maximum(
            jnp.max(a, axis=-1, keepdims=True),
            jnp.max(b, axis=-1, keepdims=True),
        )
        ea = jnp.exp(a - m)
        eb = jnp.exp(b - m)
        s = jnp.sum(ea, axis=-1, keepdims=True) + jnp.sum(eb, axis=-1, keepdims=True)
        pa = ea / s
        pb = eb / s

        @pl.when(my_z == 0)
        def _():
            o_ref[:, :VH] = pa
            o_ref[:, VH:] = pb

        @pl.when(my_z == 1)
        def _():
            o_ref[:, :VH] = pb
            o_ref[:, VH:] = pa

    return pl.pallas_call(
        body,
        grid=(T // RB,),
        in_specs=[
            pl.BlockSpec((RB, VH), lambda i: (i, 0)),
            pl.BlockSpec((RB, VH), lambda i: (i, 0)),
        ],
        out_specs=pl.BlockSpec((RB, 2 * VH), lambda i: (i, 0)),
        out_shape=jax.ShapeDtypeStruct((T, 2 * VH), jnp.float32),
    )(mine, other)


def kernel(x, W):
    logits = _gemm(x, W)
    other = _exchange(logits)
    return _softmax(logits, other)


# baseline (device time: 517899 ns/iter reference)
import jax
import jax.numpy as jnp
from jax import lax
from jax.experimental import pallas as pl
from jax.experimental.pallas import tpu as pltpu

T = 1024
D = 2048
VH = 16384
NCHUNK = 8
CW = VH // NCHUNK
RB = 128


def _gemm(x, W):

    def body(x_ref, w_ref, o_ref):
        xb = x_ref[...].astype(jnp.bfloat16)
        wb = w_ref[...].astype(jnp.bfloat16)
        acc = jnp.dot(xb, wb, preferred_element_type=jnp.float32)
        o_ref[...] = acc.astype(jnp.bfloat16)

    return pl.pallas_call(
        body,
        grid=(NCHUNK,),
        in_specs=[
            pl.BlockSpec((T, D), lambda j: (0, 0)),
            pl.BlockSpec((D, CW), lambda j: (0, j)),
        ],
        out_specs=pl.BlockSpec((T, CW), lambda j: (0, j)),
        out_shape=jax.ShapeDtypeStruct((T, VH), jnp.bfloat16),
        compiler_params=pltpu.CompilerParams(vmem_limit_bytes=100 * 1024 * 1024),
    )(x, W)


def _exchange(logits):

    def body(l_ref, o_ref, send_sem, recv_sem):
        my_x = lax.axis_index("x")
        my_y = lax.axis_index("y")
        my_z = lax.axis_index("z")
        peer = (my_x, my_y, 1 - my_z)

        barrier = pltpu.get_barrier_semaphore()
        pl.semaphore_signal(
            barrier, inc=1, device_id=peer, device_id_type=pl.DeviceIdType.MESH
        )
        pl.semaphore_wait(barrier, 1)

        rdma = pltpu.make_async_remote_copy(
            src_ref=l_ref,
            dst_ref=o_ref,
            send_sem=send_sem,
            recv_sem=recv_sem,
            device_id=peer,
            device_id_type=pl.DeviceIdType.MESH,
        )
        rdma.start()
        rdma.wait()

    return pl.pallas_call(
        body,
        out_shape=jax.ShapeDtypeStruct((T, VH), logits.dtype),
        in_specs=[pl.BlockSpec(memory_space=pl.ANY)],
        out_specs=pl.BlockSpec(memory_space=pl.ANY),
        scratch_shapes=[pltpu.SemaphoreType.DMA, pltpu.SemaphoreType.DMA],
        compiler_params=pltpu.CompilerParams(collective_id=0),
    )(logits)


def _softmax(mine, other):

    def body(a_ref, b_ref, o_ref):
        my_z = lax.axis_index("z")
        a = a_ref[...].astype(jnp.float32)
        b = b_ref[...].astype(jnp.float32)
        m = jnp.maximum(
            jnp.max(a, axis=-1, keepdims=True),
            jnp.max(b, axis=-1, keepdims=True),
        )
        ea = jnp.exp(a - m)
        eb = jnp.exp(b - m)
        s = jnp.sum(ea, axis=-1, keepdims=True) + jnp.sum(eb, axis=-1, keepdims=True)
        pa = ea / s
        pb = eb / s

        @pl.when(my_z == 0)
        def _():
            o_ref[:, :VH] = pa
            o_ref[:, VH:] = pb

        @pl.when(my_z == 1)
        def _():
            o_ref[:, :VH] = pb
            o_ref[:, VH:] = pa

    return pl.pallas_call(
        body,
        grid=(T // RB,),
        in_specs=[
            pl.BlockSpec((RB, VH), lambda i: (i, 0)),
            pl.BlockSpec((RB, VH), lambda i: (i, 0)),
        ],
        out_specs=pl.BlockSpec((RB, 2 * VH), lambda i: (i, 0)),
        out_shape=jax.ShapeDtypeStruct((T, 2 * VH), jnp.float32),
        compiler_params=pltpu.CompilerParams(vmem_limit_bytes=100 * 1024 * 1024),
    )(mine, other)


def kernel(x, W):
    logits = _gemm(x, W)
    other = _exchange(logits)
    return _softmax(logits, other)


# device time: 454222 ns/iter; 1.1402x vs baseline; 1.1402x over previous
import jax
import jax.numpy as jnp
from jax import lax
from jax.experimental import pallas as pl
from jax.experimental.pallas import tpu as pltpu

T = 1024
D = 2048
VH = 16384
NCHUNK = 8
CW = VH // NCHUNK
RB = 128


def _gemm_send(x, W):

    def body(x_ref, w_ref, o_ref, recv_ref, comm_ref, send_sems, recv_sems):
        j = pl.program_id(0)
        my_x = lax.axis_index("x")
        my_y = lax.axis_index("y")
        my_z = lax.axis_index("z")
        peer = (my_x, my_y, 1 - my_z)

        def chunk_rdma(h, slot):
            return pltpu.make_async_remote_copy(
                src_ref=comm_ref.at[slot],
                dst_ref=recv_ref.at[:, pl.ds(h * CW, CW)],
                send_sem=send_sems.at[h],
                recv_sem=recv_sems.at[h],
                device_id=peer,
                device_id_type=pl.DeviceIdType.MESH,
            )

        @pl.when(j == 0)
        def _():
            barrier = pltpu.get_barrier_semaphore()
            pl.semaphore_signal(
                barrier, inc=1, device_id=peer,
                device_id_type=pl.DeviceIdType.MESH,
            )
            pl.semaphore_wait(barrier, 1)

        @pl.when(j >= 2)
        def _():
            chunk_rdma(j - 2, j % 2).wait_send()

        xb = x_ref[...].astype(jnp.bfloat16)
        wb = w_ref[...].astype(jnp.bfloat16)
        accb = jnp.dot(xb, wb, preferred_element_type=jnp.float32).astype(
            jnp.bfloat16
        )
        o_ref[...] = accb
        comm_ref[j % 2] = accb
        chunk_rdma(j, j % 2).start()

        @pl.when(j == NCHUNK - 1)
        def _():
            for h in range(NCHUNK - 2, NCHUNK):
                chunk_rdma(h, h % 2).wait_send()
            for h in range(NCHUNK):
                chunk_rdma(h, h % 2).wait_recv()

    return pl.pallas_call(
        body,
        grid=(NCHUNK,),
        in_specs=[
            pl.BlockSpec((T, D), lambda j: (0, 0)),
            pl.BlockSpec((D, CW), lambda j: (0, j)),
        ],
        out_specs=[
            pl.BlockSpec((T, CW), lambda j: (0, j)),
            pl.BlockSpec(memory_space=pl.ANY),
        ],
        out_shape=[
            jax.ShapeDtypeStruct((T, VH), jnp.bfloat16),
            jax.ShapeDtypeStruct((T, VH), jnp.bfloat16),
        ],
        scratch_shapes=[
            pltpu.VMEM((2, T, CW), jnp.bfloat16),
            pltpu.SemaphoreType.DMA((NCHUNK,)),
            pltpu.SemaphoreType.DMA((NCHUNK,)),
        ],
        compiler_params=pltpu.CompilerParams(
            collective_id=0, vmem_limit_bytes=100 * 1024 * 1024
        ),
    )(x, W)


def _softmax(mine, other):

    def body(a_ref, b_ref, o_ref):
        my_z = lax.axis_index("z")
        a = a_ref[...].astype(jnp.float32)
        b = b_ref[...].astype(jnp.float32)
        m = jnp.maximum(
            jnp.max(a, axis=-1, keepdims=True),
            jnp.max(b, axis=-1, keepdims=True),
        )
        ea = jnp.exp(a - m)
        eb = jnp.exp(b - m)
        s = jnp.sum(ea, axis=-1, keepdims=True) + jnp.sum(eb, axis=-1, keepdims=True)
        pa = ea / s
        pb = eb / s

        @pl.when(my_z == 0)
        def _():
            o_ref[:, :VH] = pa
            o_ref[:, VH:] = pb

        @pl.when(my_z == 1)
        def _():
            o_ref[:, :VH] = pb
            o_ref[:, VH:] = pa

    return pl.pallas_call(
        body,
        grid=(T // RB,),
        in_specs=[
            pl.BlockSpec((RB, VH), lambda i: (i, 0)),
            pl.BlockSpec((RB, VH), lambda i: (i, 0)),
        ],
        out_specs=pl.BlockSpec((RB, 2 * VH), lambda i: (i, 0)),
        out_shape=jax.ShapeDtypeStruct((T, 2 * VH), jnp.float32),
        compiler_params=pltpu.CompilerParams(vmem_limit_bytes=100 * 1024 * 1024),
    )(mine, other)


def kernel(x, W):
    logits, other = _gemm_send(x, W)
    return _softmax(logits, other)


# device time: 340874 ns/iter; 1.5193x vs baseline; 1.3325x over previous
import jax
import jax.numpy as jnp
from jax import lax
from jax.experimental import pallas as pl
from jax.experimental.pallas import tpu as pltpu

T = 1024
D = 2048
VH = 16384
NCHUNK = 16
CW = VH // NCHUNK
RB = 128


NSEND = NCHUNK // 2


def _gemm_send(x, W):

    def body(x_ref, w_ref, o_ref, recv_ref, comm_ref, z_send, z_recv,
             y_send, y_recv):
        j = pl.program_id(0)
        my_x = lax.axis_index("x")
        my_y = lax.axis_index("y")
        my_z = lax.axis_index("z")
        zpeer = (my_x, my_y, 1 - my_z)
        ypeer = (my_x, 1 - my_y, my_z)

        def z_rdma(k):
            h = 2 * k + my_y
            return pltpu.make_async_remote_copy(
                src_ref=comm_ref.at[k],
                dst_ref=recv_ref.at[:, pl.ds(h * CW, CW)],
                send_sem=z_send.at[k],
                recv_sem=z_recv.at[k],
                device_id=zpeer,
                device_id_type=pl.DeviceIdType.MESH,
            )

        def y_out(k):
            h = 2 * k + my_y
            return pltpu.make_async_remote_copy(
                src_ref=recv_ref.at[:, pl.ds(h * CW, CW)],
                dst_ref=recv_ref.at[:, pl.ds(h * CW, CW)],
                send_sem=y_send.at[k],
                recv_sem=y_recv.at[k],
                device_id=ypeer,
                device_id_type=pl.DeviceIdType.MESH,
            )

        def y_in(k):
            h = 2 * k + (1 - my_y)
            return pltpu.make_async_remote_copy(
                src_ref=recv_ref.at[:, pl.ds(h * CW, CW)],
                dst_ref=recv_ref.at[:, pl.ds(h * CW, CW)],
                send_sem=y_send.at[k],
                recv_sem=y_recv.at[k],
                device_id=ypeer,
                device_id_type=pl.DeviceIdType.MESH,
            )

        @pl.when(j == 0)
        def _():
            barrier = pltpu.get_barrier_semaphore()
            for peer in (zpeer, ypeer):
                pl.semaphore_signal(
                    barrier, inc=1, device_id=peer,
                    device_id_type=pl.DeviceIdType.MESH,
                )
            pl.semaphore_wait(barrier, 2)

        xb = x_ref[...].astype(jnp.bfloat16)
        wb = w_ref[...].astype(jnp.bfloat16)
        accb = jnp.dot(xb, wb, preferred_element_type=jnp.float32).astype(
            jnp.bfloat16
        )
        o_ref[...] = accb

        @pl.when(j % 2 == my_y)
        def _():
            k = j // 2
            comm_ref[k] = accb
            z_rdma(k).start()

        @pl.when(j == NCHUNK - 1)
        def _():
            for k in range(NSEND):
                z_rdma(k).wait_recv()
                y_out(k).start()
            for k in range(NSEND):
                z_rdma(k).wait_send()
            for k in range(NSEND):
                y_out(k).wait_send()
                y_in(k).wait_recv()

    return pl.pallas_call(
        body,
        grid=(NCHUNK,),
        in_specs=[
            pl.BlockSpec((T, D), lambda j: (0, 0)),
            pl.BlockSpec((D, CW), lambda j: (0, j)),
        ],
        out_specs=[
            pl.BlockSpec((T, CW), lambda j: (0, j)),
            pl.BlockSpec(memory_space=pl.ANY),
        ],
        out_shape=[
            jax.ShapeDtypeStruct((T, VH), jnp.bfloat16),
            jax.ShapeDtypeStruct((T, VH), jnp.bfloat16),
        ],
        scratch_shapes=[
            pltpu.VMEM((NSEND, T, CW), jnp.bfloat16),
            pltpu.SemaphoreType.DMA((NSEND,)),
            pltpu.SemaphoreType.DMA((NSEND,)),
            pltpu.SemaphoreType.DMA((NSEND,)),
            pltpu.SemaphoreType.DMA((NSEND,)),
        ],
        compiler_params=pltpu.CompilerParams(
            collective_id=0, vmem_limit_bytes=100 * 1024 * 1024
        ),
    )(x, W)


def _softmax(mine, other):

    def body(a_ref, b_ref, o_ref):
        my_z = lax.axis_index("z")
        a = a_ref[...].astype(jnp.float32)
        b = b_ref[...].astype(jnp.float32)
        m = jnp.maximum(
            jnp.max(a, axis=-1, keepdims=True),
            jnp.max(b, axis=-1, keepdims=True),
        )
        ea = jnp.exp(a - m)
        eb = jnp.exp(b - m)
        s = jnp.sum(ea, axis=-1, keepdims=True) + jnp.sum(eb, axis=-1, keepdims=True)
        pa = ea / s
        pb = eb / s

        @pl.when(my_z == 0)
        def _():
            o_ref[:, :VH] = pa
            o_ref[:, VH:] = pb

        @pl.when(my_z == 1)
        def _():
            o_ref[:, :VH] = pb
            o_ref[:, VH:] = pa

    return pl.pallas_call(
        body,
        grid=(T // RB,),
        in_specs=[
            pl.BlockSpec((RB, VH), lambda i: (i, 0)),
            pl.BlockSpec((RB, VH), lambda i: (i, 0)),
        ],
        out_specs=pl.BlockSpec((RB, 2 * VH), lambda i: (i, 0)),
        out_shape=jax.ShapeDtypeStruct((T, 2 * VH), jnp.float32),
        compiler_params=pltpu.CompilerParams(vmem_limit_bytes=100 * 1024 * 1024),
    )(mine, other)


def kernel(x, W):
    logits, other = _gemm_send(x, W)
    return _softmax(logits, other)


# device time: 308325 ns/iter; 1.6797x vs baseline; 1.1056x over previous
import jax
import jax.numpy as jnp
from jax import lax
from jax.experimental import pallas as pl
from jax.experimental.pallas import tpu as pltpu

T = 1024
D = 2048
VH = 16384
NCHUNK = 16
CW = VH // NCHUNK
RB = 128


NSEND = NCHUNK // 2


def _gemm_send(x, W):

    def body(x_ref, w_ref, o_ref, recv_ref, comm_ref, z_send, z_recv,
             y_send, y_recv):
        j = pl.program_id(0)
        my_x = lax.axis_index("x")
        my_y = lax.axis_index("y")
        my_z = lax.axis_index("z")
        zpeer = (my_x, my_y, 1 - my_z)
        ypeer = (my_x, 1 - my_y, my_z)

        def z_rdma(k):
            h = 2 * k + my_y
            return pltpu.make_async_remote_copy(
                src_ref=comm_ref.at[k],
                dst_ref=recv_ref.at[:, pl.ds(h * CW, CW)],
                send_sem=z_send.at[k],
                recv_sem=z_recv.at[k],
                device_id=zpeer,
                device_id_type=pl.DeviceIdType.MESH,
            )

        def y_out(k):
            h = 2 * k + my_y
            return pltpu.make_async_remote_copy(
                src_ref=recv_ref.at[:, pl.ds(h * CW, CW)],
                dst_ref=recv_ref.at[:, pl.ds(h * CW, CW)],
                send_sem=y_send.at[k],
                recv_sem=y_recv.at[k],
                device_id=ypeer,
                device_id_type=pl.DeviceIdType.MESH,
            )

        def y_in(k):
            h = 2 * k + (1 - my_y)
            return pltpu.make_async_remote_copy(
                src_ref=recv_ref.at[:, pl.ds(h * CW, CW)],
                dst_ref=recv_ref.at[:, pl.ds(h * CW, CW)],
                send_sem=y_send.at[k],
                recv_sem=y_recv.at[k],
                device_id=ypeer,
                device_id_type=pl.DeviceIdType.MESH,
            )

        @pl.when(j == 0)
        def _():
            barrier = pltpu.get_barrier_semaphore()
            for peer in (zpeer, ypeer):
                pl.semaphore_signal(
                    barrier, inc=1, device_id=peer,
                    device_id_type=pl.DeviceIdType.MESH,
                )
            pl.semaphore_wait(barrier, 2)

        xb = x_ref[...].astype(jnp.bfloat16)
        wb = w_ref[...].astype(jnp.bfloat16)
        accb = jnp.dot(xb, wb, preferred_element_type=jnp.float32).astype(
            jnp.bfloat16
        )
        o_ref[...] = accb

        @pl.when(j % 2 == my_y)
        def _():
            k = j // 2
            comm_ref[k] = accb
            z_rdma(k).start()

        @pl.when(j >= NSEND)
        def _():
            k = j - NSEND
            z_rdma(k).wait_recv()
            y_out(k).start()

        @pl.when(j == NCHUNK - 1)
        def _():
            for k in range(NSEND):
                z_rdma(k).wait_send()
            for k in range(NSEND):
                y_out(k).wait_send()
                y_in(k).wait_recv()

    return pl.pallas_call(
        body,
        grid=(NCHUNK,),
        in_specs=[
            pl.BlockSpec((T, D), lambda j: (0, 0)),
            pl.BlockSpec((D, CW), lambda j: (0, j)),
        ],
        out_specs=[
            pl.BlockSpec((T, CW), lambda j: (0, j)),
            pl.BlockSpec(memory_space=pl.ANY),
        ],
        out_shape=[
            jax.ShapeDtypeStruct((T, VH), jnp.bfloat16),
            jax.ShapeDtypeStruct((T, VH), jnp.bfloat16),
        ],
        scratch_shapes=[
            pltpu.VMEM((NSEND, T, CW), jnp.bfloat16),
            pltpu.SemaphoreType.DMA((NSEND,)),
            pltpu.SemaphoreType.DMA((NSEND,)),
            pltpu.SemaphoreType.DMA((NSEND,)),
            pltpu.SemaphoreType.DMA((NSEND,)),
        ],
        compiler_params=pltpu.CompilerParams(
            collective_id=0, vmem_limit_bytes=100 * 1024 * 1024
        ),
    )(x, W)


def _softmax(mine, other):

    def body(a_ref, b_ref, o_ref):
        my_z = lax.axis_index("z")
        a = a_ref[...].astype(jnp.float32)
        b = b_ref[...].astype(jnp.float32)
        m = jnp.maximum(
            jnp.max(a, axis=-1, keepdims=True),
            jnp.max(b, axis=-1, keepdims=True),
        )
        ea = jnp.exp(a - m)
        eb = jnp.exp(b - m)
        s = jnp.sum(ea, axis=-1, keepdims=True) + jnp.sum(eb, axis=-1, keepdims=True)
        pa = ea / s
        pb = eb / s

        @pl.when(my_z == 0)
        def _():
            o_ref[:, :VH] = pa
            o_ref[:, VH:] = pb

        @pl.when(my_z == 1)
        def _():
            o_ref[:, :VH] = pb
            o_ref[:, VH:] = pa

    return pl.pallas_call(
        body,
        grid=(T // RB,),
        in_specs=[
            pl.BlockSpec((RB, VH), lambda i: (i, 0)),
            pl.BlockSpec((RB, VH), lambda i: (i, 0)),
        ],
        out_specs=pl.BlockSpec((RB, 2 * VH), lambda i: (i, 0)),
        out_shape=jax.ShapeDtypeStruct((T, 2 * VH), jnp.float32),
        compiler_params=pltpu.CompilerParams(vmem_limit_bytes=100 * 1024 * 1024),
    )(mine, other)


def kernel(x, W):
    logits, other = _gemm_send(x, W)
    return _softmax(logits, other)


# device time: 69583 ns/iter; 7.4429x vs baseline; 4.4310x over previous
import jax
import jax.numpy as jnp
from jax import lax
from jax.experimental import pallas as pl
from jax.experimental.pallas import tpu as pltpu

T = 1024
D = 2048
VH = 16384
NCHUNK = 16
CW = VH // NCHUNK
RB = 128


NQ = 4
NM = NCHUNK // NQ


def _gemm_send(x, W):

    def body(x_ref, w_ref, o_ref, recv_ref, comm_ref,
             z_send, z_recv, xd_send, xd_recv, yd_send, yd_recv,
             xt_send, xt_recv, yt_send, yt_recv):
        j = pl.program_id(0)
        my_x = lax.axis_index("x")
        my_y = lax.axis_index("y")
        my_z = lax.axis_index("z")
        zpeer = (my_x, my_y, 1 - my_z)
        xpeer = (1 - my_x, my_y, my_z)
        ypeer = (my_x, 1 - my_y, my_z)

        q_me = my_x + 2 * my_y
        q_xp = (1 - my_x) + 2 * my_y
        q_yp = my_x + 2 * (1 - my_y)
        q_d = (1 - my_x) + 2 * (1 - my_y)

        def cols(h):
            return recv_ref.at[:, pl.ds(h * CW, CW)]

        def rdma(src, h, send_sem, recv_sem, peer):
            return pltpu.make_async_remote_copy(
                src_ref=src,
                dst_ref=cols(h),
                send_sem=send_sem,
                recv_sem=recv_sem,
                device_id=peer,
                device_id_type=pl.DeviceIdType.MESH,
            )

        def z_rdma(m):
            return rdma(comm_ref.at[m], 4 * m + q_me,
                        z_send.at[m], z_recv.at[m], zpeer)

        def xd_out(m):
            h = 4 * m + q_me
            return rdma(cols(h), h, xd_send.at[m], xd_recv.at[m], xpeer)

        def yd_out(m):
            h = 4 * m + q_me
            return rdma(cols(h), h, yd_send.at[m], yd_recv.at[m], ypeer)

        def xd_in(m):
            h = 4 * m + q_xp
            return rdma(cols(h), h, xd_send.at[m], xd_recv.at[m], xpeer)

        def yd_in(m):
            h = 4 * m + q_yp
            return rdma(cols(h), h, yd_send.at[m], yd_recv.at[m], ypeer)

        def yt_out(m):
            h = 4 * m + q_xp
            return rdma(cols(h), h, yt_send.at[m], yt_recv.at[m], ypeer)

        def yt_in(m):
            h = 4 * m + q_d
            return rdma(cols(h), h, yt_send.at[m], yt_recv.at[m], ypeer)

        def xt_out(m):
            h = 4 * m + q_yp
            return rdma(cols(h), h, xt_send.at[m - 2], xt_recv.at[m - 2], xpeer)

        def xt_in(m):
            h = 4 * m + q_d
            return rdma(cols(h), h, xt_send.at[m - 2], xt_recv.at[m - 2], xpeer)

        @pl.when(j == 0)
        def _():
            barrier = pltpu.get_barrier_semaphore()
            for peer in (zpeer, xpeer, ypeer):
                pl.semaphore_signal(
                    barrier, inc=1, device_id=peer,
                    device_id_type=pl.DeviceIdType.MESH,
                )
            pl.semaphore_wait(barrier, 3)

        xb = x_ref[...].astype(jnp.bfloat16)
        wb = w_ref[...].astype(jnp.bfloat16)
        accb = jnp.dot(xb, wb, preferred_element_type=jnp.float32).astype(
            jnp.bfloat16
        )
        o_ref[...] = accb

        @pl.when(j % NQ == q_me)
        def _():
            m = j // NQ
            comm_ref[m] = accb
            z_rdma(m).start()

        @pl.when((j >= 6) & ((j - 6) % 3 == 0))
        def _():
            m = (j - 6) // 3
            z_rdma(m).wait_recv()
            xd_out(m).start()
            yd_out(m).start()

        @pl.when(j == NCHUNK - 1)
        def _():
            for m in (0, 1):
                xd_in(m).wait_recv()
                yt_out(m).start()
            for m in (2, 3):
                yd_in(m).wait_recv()
                xt_out(m).start()
            for m in (2, 3):
                xd_in(m).wait_recv()
            for m in (0, 1):
                yd_in(m).wait_recv()
            for m in (0, 1):
                yt_in(m).wait_recv()
            for m in (2, 3):
                xt_in(m).wait_recv()
            for m in range(NM):
                z_rdma(m).wait_send()
                xd_out(m).wait_send()
                yd_out(m).wait_send()
            for m in (0, 1):
                yt_out(m).wait_send()
            for m in (2, 3):
                xt_out(m).wait_send()

    return pl.pallas_call(
        body,
        grid=(NCHUNK,),
        in_specs=[
            pl.BlockSpec((T, D), lambda j: (0, 0)),
            pl.BlockSpec((D, CW), lambda j: (0, j)),
        ],
        out_specs=[
            pl.BlockSpec((T, CW), lambda j: (0, j)),
            pl.BlockSpec(memory_space=pl.ANY),
        ],
        out_shape=[
            jax.ShapeDtypeStruct((T, VH), jnp.bfloat16),
            jax.ShapeDtypeStruct((T, VH), jnp.bfloat16),
        ],
        scratch_shapes=[
            pltpu.VMEM((NM, T, CW), jnp.bfloat16),
            pltpu.SemaphoreType.DMA((NM,)),
            pltpu.SemaphoreType.DMA((NM,)),
            pltpu.SemaphoreType.DMA((NM,)),
            pltpu.SemaphoreType.DMA((NM,)),
            pltpu.SemaphoreType.DMA((NM,)),
            pltpu.SemaphoreType.DMA((NM,)),
            pltpu.SemaphoreType.DMA((2,)),
            pltpu.SemaphoreType.DMA((2,)),
            pltpu.SemaphoreType.DMA((2,)),
            pltpu.SemaphoreType.DMA((2,)),
        ],
        compiler_params=pltpu.CompilerParams(
            collective_id=0, vmem_limit_bytes=100 * 1024 * 1024
        ),
    )(x, W)


def _softmax(mine, other):

    def body(a_ref, b_ref, o_ref):
        my_z = lax.axis_index("z")
        a = a_ref[...].astype(jnp.float32)
        b = b_ref[...].astype(jnp.float32)
        m = jnp.maximum(
            jnp.max(a, axis=-1, keepdims=True),
            jnp.max(b, axis=-1, keepdims=True),
        )
        ea = jnp.exp(a - m)
        eb = jnp.exp(b - m)
        s = jnp.sum(ea, axis=-1, keepdims=True) + jnp.sum(eb, axis=-1, keepdims=True)
        pa = ea / s
        pb = eb / s

        @pl.when(my_z == 0)
        def _():
            o_ref[:, :VH] = pa
            o_ref[:, VH:] = pb

        @pl.when(my_z == 1)
        def _():
            o_ref[:, :VH] = pb
            o_ref[:, VH:] = pa

    return pl.pallas_call(
        body,
        grid=(T // RB,),
        in_specs=[
            pl.BlockSpec((RB, VH), lambda i: (i, 0)),
            pl.BlockSpec((RB, VH), lambda i: (i, 0)),
        ],
        out_specs=pl.BlockSpec((RB, 2 * VH), lambda i: (i, 0)),
        out_shape=jax.ShapeDtypeStruct((T, 2 * VH), jnp.float32),
        compiler_params=pltpu.CompilerParams(vmem_limit_bytes=100 * 1024 * 1024),
    )(mine, other)


def kernel(x, W):
    logits, other = _gemm_send(x, W)
    return _softmax(logits, other)
